# Initial kernel scaffold; baseline (speedup 1.0000x reference)
import jax
import jax.numpy as jnp
from jax.experimental import pallas as pl


def kernel(cov, embeddings, noise_scale):
    raise NotImplementedError("write your pallas kernel here")


# speedup vs baseline: 3.2246x; 3.2246x over previous
import jax
import jax.numpy as jnp
from jax.experimental import pallas as pl
from jax.experimental.pallas import tpu as pltpu

_B = 8
_N = 2048
_R = 256
_NR = _N // _R


def _diag_body(ns_ref, cov_ref, het_ref, out_ref):
    r = pl.program_id(1)
    r0 = r * _R
    row = jax.lax.broadcasted_iota(jnp.int32, (_R, _N), 0)
    col = jax.lax.broadcasted_iota(jnp.int32, (_R, _N), 1)
    ens = jnp.exp(ns_ref[0])
    val = jnp.exp(het_ref[...]) + ens
    out_ref[0] = cov_ref[0] + jnp.where(col == row + r0, val, 0.0)


def kernel(cov, embeddings, noise_scale):
    het = embeddings[:, :, -1].reshape(_B * _N, 1)
    grid = (_B, _NR)
    out = pl.pallas_call(
        _diag_body,
        grid=grid,
        in_specs=[
            pl.BlockSpec(memory_space=pltpu.SMEM),
            pl.BlockSpec((1, _R, _N), lambda b, r: (b, r, 0)),
            pl.BlockSpec((_R, 1), lambda b, r: (b * _NR + r, 0)),
        ],
        out_specs=pl.BlockSpec((1, _R, _N), lambda b, r: (b, r, 0)),
        out_shape=jax.ShapeDtypeStruct((_B, _N, _N), jnp.float32),
    )(noise_scale, cov, het)
    return out


# speedup vs baseline: 3.5570x; 1.1031x over previous
import jax
import jax.numpy as jnp
from jax.experimental import pallas as pl
from jax.experimental.pallas import tpu as pltpu

_B = 8
_N = 2048
_R = 512
_NR = _N // _R


def _diag_body(ns_ref, cov_ref, het_ref, out_ref):
    r = pl.program_id(1)
    r0 = r * _R
    row = jax.lax.broadcasted_iota(jnp.int32, (_R, _N), 0)
    col = jax.lax.broadcasted_iota(jnp.int32, (_R, _N), 1)
    ens = jnp.exp(ns_ref[0])
    val = jnp.exp(het_ref[...]) + ens
    out_ref[0] = cov_ref[0] + jnp.where(col == row + r0, val, 0.0)


def kernel(cov, embeddings, noise_scale):
    het = embeddings[:, :, -1].reshape(_B * _N, 1)
    grid = (_B, _NR)
    out = pl.pallas_call(
        _diag_body,
        grid=grid,
        in_specs=[
            pl.BlockSpec(memory_space=pltpu.SMEM),
            pl.BlockSpec((1, _R, _N), lambda b, r: (b, r, 0)),
            pl.BlockSpec((_R, 1), lambda b, r: (b * _NR + r, 0)),
        ],
        out_specs=pl.BlockSpec((1, _R, _N), lambda b, r: (b, r, 0)),
        out_shape=jax.ShapeDtypeStruct((_B, _N, _N), jnp.float32),
    )(noise_scale, cov, het)
    return out


# speedup vs baseline: 3.6111x; 1.0152x over previous
import jax
import jax.numpy as jnp
from jax.experimental import pallas as pl
from jax.experimental.pallas import tpu as pltpu

_B = 8
_N = 2048
_R = 1024
_NR = _N // _R


def _diag_body(ns_ref, cov_ref, het_ref, out_ref):
    r = pl.program_id(1)
    r0 = r * _R
    row = jax.lax.broadcasted_iota(jnp.int32, (_R, _N), 0)
    col = jax.lax.broadcasted_iota(jnp.int32, (_R, _N), 1)
    ens = jnp.exp(ns_ref[0])
    val = jnp.exp(het_ref[...]) + ens
    out_ref[0] = cov_ref[0] + jnp.where(col == row + r0, val, 0.0)


def kernel(cov, embeddings, noise_scale):
    het = embeddings[:, :, -1].reshape(_B * _N, 1)
    grid = (_B, _NR)
    out = pl.pallas_call(
        _diag_body,
        grid=grid,
        in_specs=[
            pl.BlockSpec(memory_space=pltpu.SMEM),
            pl.BlockSpec((1, _R, _N), lambda b, r: (b, r, 0)),
            pl.BlockSpec((_R, 1), lambda b, r: (b * _NR + r, 0)),
        ],
        out_specs=pl.BlockSpec((1, _R, _N), lambda b, r: (b, r, 0)),
        out_shape=jax.ShapeDtypeStruct((_B, _N, _N), jnp.float32),
    )(noise_scale, cov, het)
    return out
